# baseline (speedup 1.0000x reference)
import dataclasses
import functools

import jax
import jax.numpy as jnp
from jax import lax
from jax.experimental import pallas as pl
from jax.experimental.pallas import tpu as pltpu
from jax.experimental.pallas import tpu_sc as plsc

N_USERS = 25000
N_ITEMS = 25000
N = N_USERS + N_ITEMS
D = 32
N_LAYERS = 3
NNZ = 1600000

NC = 2
NS = 16
L = 16
NW = NC * NS

CHUNK = 128
NNZ_PAD = 1638400
CROWS = NNZ_PAD // CHUNK
CROWS_PER_W = CROWS // NW
K = 16
NUM_BLOCKS = CROWS_PER_W // K
NBUF = 4
N_PAD = 50048
ROWS_PER_SUB = N_PAD // NS

_MESH = plsc.VectorSubcoreMesh(core_axis_name="c", subcore_axis_name="s")

_CP = pltpu.CompilerParams(use_tc_tiling_on_sc=False)
if "needs_layout_passes" in pltpu.CompilerParams.__dataclass_fields__:
    _CP = dataclasses.replace(_CP, needs_layout_passes=False)


def _layer_body(row_hbm, col_hbm, val_hbm, emb_hbm, out_hbm,
                cb0, cb1, rb0, rb1, vb0, vb1, g0, g1, g2, g3, acc,
                gs0, gs1, gs2, gs3, ss0, ss1, ss2, ss3,
                is0, is1, zsem):
    cid = lax.axis_index("c")
    sid = lax.axis_index("s")
    wid = cid * NS + sid

    colb = (cb0, cb1)
    rowb = (rb0, rb1)
    valb = (vb0, vb1)
    gbuf = (g0, g1, g2, g3)
    gsem = (gs0, gs1, gs2, gs3)
    ssem = (ss0, ss1, ss2, ss3)
    isem = (is0, is1)

    cb00 = wid * CROWS_PER_W
    pltpu.async_copy(row_hbm.at[pl.ds(cb00, K)], rowb[0], isem[0])
    pltpu.async_copy(col_hbm.at[pl.ds(cb00, K)], colb[0], isem[0])
    pltpu.async_copy(val_hbm.at[pl.ds(cb00, K)], valb[0], isem[0])

    zero = jnp.zeros((L,), jnp.float32)
    for b in range(NBUF):
        g = gbuf[b]

        @pl.loop(0, CHUNK)
        def _(i):
            g[i, pl.ds(0, L)] = zero
            g[i, pl.ds(L, L)] = zero

    abase = sid * ROWS_PER_SUB
    nz = ROWS_PER_SUB // CHUNK
    for k in range(nz):
        pltpu.async_copy(gbuf[k % NBUF],
                         acc.at[pl.ds(abase + k * CHUNK, CHUNK)], zsem)
    rem = ROWS_PER_SUB - nz * CHUNK
    pltpu.async_copy(gbuf[0].at[pl.ds(0, rem)],
                     acc.at[pl.ds(abase + nz * CHUNK, rem)], zsem)
    for k in range(nz):
        pltpu.make_async_copy(gbuf[k % NBUF],
                              acc.at[pl.ds(abase, CHUNK)], zsem).wait()
    pltpu.make_async_copy(gbuf[0].at[pl.ds(0, rem)],
                          acc.at[pl.ds(abase, rem)], zsem).wait()

    plsc.subcore_barrier()

    def gissue(p, c, b):
        pltpu.async_copy(emb_hbm.at[colb[p].at[c]], gbuf[b], gsem[b])

    def gwait(b):
        pltpu.make_async_copy(emb_hbm.at[colb[0].at[0]], gbuf[b],
                              gsem[b]).wait()

    def sissue(p, c, b):
        pltpu.async_copy(gbuf[b], acc.at[rowb[p].at[c]], ssem[b], add=True)

    def swait(b):
        pltpu.make_async_copy(gbuf[b], acc.at[rowb[0].at[0]], ssem[b]).wait()

    def iissue(p, blk):
        cb = wid * CROWS_PER_W + blk * K
        pltpu.async_copy(row_hbm.at[pl.ds(cb, K)], rowb[p], isem[p])
        pltpu.async_copy(col_hbm.at[pl.ds(cb, K)], colb[p], isem[p])
        pltpu.async_copy(val_hbm.at[pl.ds(cb, K)], valb[p], isem[p])

    def iwait(p):
        pltpu.make_async_copy(row_hbm.at[pl.ds(0, K)], rowb[p], isem[p]).wait()
        pltpu.make_async_copy(col_hbm.at[pl.ds(0, K)], colb[p], isem[p]).wait()
        pltpu.make_async_copy(val_hbm.at[pl.ds(0, K)], valb[p], isem[p]).wait()

    def mul(p, c, b):
        g = gbuf[b]
        vb = valb[p]

        @plsc.parallel_loop(0, CHUNK, step=L, unroll=2)
        def _(e0):
            vv = vb[c, pl.ds(e0, L)]
            for i in range(L):
                v = vv.at[jnp.full((L,), i, jnp.int32)].get(
                    mode="promise_in_bounds")
                e = e0 + i
                g[e, pl.ds(0, L)] = g[e, pl.ds(0, L)] * v
                g[e, pl.ds(L, L)] = g[e, pl.ds(L, L)] * v

    def body(blk, p, last):
        if not last:
            iissue(1 - p, blk + 1)

        @pl.loop(0, K - NBUF, step=NBUF)
        def _(c0):
            for b in range(NBUF):
                gwait(b)
                mul(p, c0 + b, b)
                sissue(p, c0 + b, b)
            for b in range(NBUF):
                swait(b)
                gissue(p, c0 + NBUF + b, b)

        for b in range(NBUF):
            gwait(b)
            mul(p, K - NBUF + b, b)
            sissue(p, K - NBUF + b, b)
        if not last:
            iwait(1 - p)
            for b in range(NBUF):
                swait(b)
                gissue(1 - p, b, b)
        else:
            for b in range(NBUF):
                swait(b)

    iwait(0)
    for b in range(NBUF):
        gissue(0, b, b)

    @pl.loop(0, NUM_BLOCKS - 1, step=2)
    def _(blk):
        body(blk, 0, False)
        body(blk + 1, 1, False)

    body(NUM_BLOCKS - 1, 0, True)

    plsc.subcore_barrier()

    pltpu.sync_copy(acc.at[pl.ds(sid * ROWS_PER_SUB, ROWS_PER_SUB)],
                    out_hbm.at[cid].at[pl.ds(sid * ROWS_PER_SUB, ROWS_PER_SUB)])


@functools.partial(
    pl.kernel,
    out_type=jax.ShapeDtypeStruct((NC, N_PAD, D), jnp.float32),
    mesh=_MESH,
    scratch_types=(
        [pltpu.VMEM((K, CHUNK), jnp.int32)] * 2
        + [pltpu.VMEM((K, CHUNK), jnp.int32)] * 2
        + [pltpu.VMEM((K, CHUNK), jnp.float32)] * 2
        + [pltpu.VMEM((CHUNK, D), jnp.float32)] * NBUF
        + [pltpu.VMEM_SHARED((N_PAD, D), jnp.float32)]
        + [pltpu.SemaphoreType.DMA] * (2 * NBUF + 3)
    ),
    compiler_params=_CP,
)
def _spmm_layer(row_hbm, col_hbm, val_hbm, emb_hbm, out_hbm, *scratch):
    _layer_body(row_hbm, col_hbm, val_hbm, emb_hbm, out_hbm, *scratch)


def kernel(adj_indices, adj_values, user_emb, item_emb):
    pad = NNZ_PAD - NNZ
    row = jnp.concatenate(
        [adj_indices[0], jnp.full((pad,), N, jnp.int32)]).reshape(CROWS, CHUNK)
    col = jnp.concatenate(
        [adj_indices[1], jnp.zeros((pad,), jnp.int32)]).reshape(CROWS, CHUNK)
    val = jnp.concatenate(
        [adj_values, jnp.zeros((pad,), jnp.float32)]).reshape(CROWS, CHUNK)
    emb = jnp.concatenate(
        [user_emb, item_emb, jnp.zeros((N_PAD - N, D), jnp.float32)], axis=0)

    total = emb
    cur = emb
    for _ in range(N_LAYERS):
        partials = _spmm_layer(row, col, val, cur)
        cur = partials[0] + partials[1]
        total = total + cur

    final = total * (1.0 / (N_LAYERS + 1))
    return final[:N_USERS], final[N_USERS:N]


# speedup vs baseline: 2.1168x; 2.1168x over previous
import dataclasses
import functools

import jax
import jax.numpy as jnp
from jax import lax
from jax.experimental import pallas as pl
from jax.experimental.pallas import tpu as pltpu
from jax.experimental.pallas import tpu_sc as plsc

N_USERS = 25000
N_ITEMS = 25000
N = N_USERS + N_ITEMS
D = 32
N_LAYERS = 3
NNZ = 1600000

NC = 2
NS = 16
L = 16
NW = NC * NS

CHUNK = 80
CROWS = NNZ // CHUNK
CROWS_PER_W = CROWS // NW
K = 25
NUM_BLOCKS = CROWS_PER_W // K
NBUF = 5
N_PAD = 50048
ROWS_PER_SUB = N_PAD // NS

_MESH = plsc.VectorSubcoreMesh(core_axis_name="c", subcore_axis_name="s")

_CP = pltpu.CompilerParams(use_tc_tiling_on_sc=False)
if "needs_layout_passes" in pltpu.CompilerParams.__dataclass_fields__:
    _CP = dataclasses.replace(_CP, needs_layout_passes=False)


def _layer_body(row_hbm, col_hbm, val_hbm, emb_hbm, out_hbm,
                cb0, cb1, rb0, rb1, vb0, vb1, g0, g1, g2, g3, g4, acc,
                gs0, gs1, gs2, gs3, gs4, ss0, ss1, ss2, ss3, ss4,
                is0, is1, zsem):
    cid = lax.axis_index("c")
    sid = lax.axis_index("s")
    wid = cid * NS + sid

    colb = (cb0, cb1)
    rowb = (rb0, rb1)
    valb = (vb0, vb1)
    gbuf = (g0, g1, g2, g3, g4)
    gsem = (gs0, gs1, gs2, gs3, gs4)
    ssem = (ss0, ss1, ss2, ss3, ss4)
    isem = (is0, is1)

    cb00 = wid * CROWS_PER_W
    pltpu.async_copy(row_hbm.at[pl.ds(cb00, K)], rowb[0], isem[0])
    pltpu.async_copy(col_hbm.at[pl.ds(cb00, K)], colb[0], isem[0])
    pltpu.async_copy(val_hbm.at[pl.ds(cb00, K)], valb[0], isem[0])

    zero = jnp.zeros((L,), jnp.float32)
    for b in range(NBUF):
        g = gbuf[b]

        @pl.loop(0, CHUNK)
        def _(i):
            g[i, pl.ds(0, L)] = zero
            g[i, pl.ds(L, L)] = zero

    abase = sid * ROWS_PER_SUB
    nz = ROWS_PER_SUB // CHUNK
    for k in range(nz):
        pltpu.async_copy(gbuf[k % NBUF],
                         acc.at[pl.ds(abase + k * CHUNK, CHUNK)], zsem)
    rem = ROWS_PER_SUB - nz * CHUNK
    pltpu.async_copy(gbuf[0].at[pl.ds(0, rem)],
                     acc.at[pl.ds(abase + nz * CHUNK, rem)], zsem)
    for k in range(nz):
        pltpu.make_async_copy(gbuf[k % NBUF],
                              acc.at[pl.ds(abase, CHUNK)], zsem).wait()
    pltpu.make_async_copy(gbuf[0].at[pl.ds(0, rem)],
                          acc.at[pl.ds(abase, rem)], zsem).wait()

    plsc.subcore_barrier()

    def gissue(p, c, b):
        pltpu.async_copy(emb_hbm.at[colb[p].at[c]], gbuf[b], gsem[b])

    def gwait(b):
        pltpu.make_async_copy(emb_hbm.at[colb[0].at[0]], gbuf[b],
                              gsem[b]).wait()

    def sissue(p, c, b):
        pltpu.async_copy(gbuf[b], acc.at[rowb[p].at[c]], ssem[b], add=True)

    def swait(b):
        pltpu.make_async_copy(gbuf[b], acc.at[rowb[0].at[0]], ssem[b]).wait()

    def iissue(p, blk):
        cb = wid * CROWS_PER_W + blk * K
        pltpu.async_copy(row_hbm.at[pl.ds(cb, K)], rowb[p], isem[p])
        pltpu.async_copy(col_hbm.at[pl.ds(cb, K)], colb[p], isem[p])
        pltpu.async_copy(val_hbm.at[pl.ds(cb, K)], valb[p], isem[p])

    def iwait(p):
        pltpu.make_async_copy(row_hbm.at[pl.ds(0, K)], rowb[p], isem[p]).wait()
        pltpu.make_async_copy(col_hbm.at[pl.ds(0, K)], colb[p], isem[p]).wait()
        pltpu.make_async_copy(val_hbm.at[pl.ds(0, K)], valb[p], isem[p]).wait()

    def mul(p, c, b):
        g = gbuf[b]
        vb = valb[p]

        @plsc.parallel_loop(0, CHUNK, step=L, unroll=4)
        def _(e0):
            vv = vb[c, pl.ds(e0, L)]
            for i in range(L):
                v = vv.at[jnp.full((L,), i, jnp.int32)].get(
                    mode="promise_in_bounds")
                e = e0 + i
                g[e, pl.ds(0, L)] = g[e, pl.ds(0, L)] * v
                g[e, pl.ds(L, L)] = g[e, pl.ds(L, L)] * v

    def body(blk, p, last):
        if not last:
            iissue(1 - p, blk + 1)

        @pl.loop(0, K - NBUF, step=NBUF)
        def _(c0):
            for b in range(NBUF):
                gwait(b)
                mul(p, c0 + b, b)
                sissue(p, c0 + b, b)
            for b in range(NBUF):
                swait(b)
                gissue(p, c0 + NBUF + b, b)

        for b in range(NBUF):
            gwait(b)
            mul(p, K - NBUF + b, b)
            sissue(p, K - NBUF + b, b)
        if not last:
            iwait(1 - p)
            for b in range(NBUF):
                swait(b)
                gissue(1 - p, b, b)
        else:
            for b in range(NBUF):
                swait(b)

    iwait(0)
    for b in range(NBUF):
        gissue(0, b, b)

    @pl.loop(0, NUM_BLOCKS - 1, step=2)
    def _(blk):
        body(blk, 0, False)
        body(blk + 1, 1, False)

    body(NUM_BLOCKS - 1, 0, True)

    plsc.subcore_barrier()

    pltpu.sync_copy(acc.at[pl.ds(sid * ROWS_PER_SUB, ROWS_PER_SUB)],
                    out_hbm.at[cid].at[pl.ds(sid * ROWS_PER_SUB, ROWS_PER_SUB)])


@functools.partial(
    pl.kernel,
    out_type=jax.ShapeDtypeStruct((NC, N_PAD, D), jnp.float32),
    mesh=_MESH,
    scratch_types=(
        [pltpu.VMEM((K, CHUNK), jnp.int32)] * 2
        + [pltpu.VMEM((K, CHUNK), jnp.int32)] * 2
        + [pltpu.VMEM((K, CHUNK), jnp.float32)] * 2
        + [pltpu.VMEM((CHUNK, D), jnp.float32)] * NBUF
        + [pltpu.VMEM_SHARED((N_PAD, D), jnp.float32)]
        + [pltpu.SemaphoreType.DMA] * (2 * NBUF + 3)
    ),
    compiler_params=_CP,
)
def _spmm_layer(row_hbm, col_hbm, val_hbm, emb_hbm, out_hbm, *scratch):
    _layer_body(row_hbm, col_hbm, val_hbm, emb_hbm, out_hbm, *scratch)


def kernel(adj_indices, adj_values, user_emb, item_emb):
    row = adj_indices[0].reshape(CROWS, CHUNK)
    col = adj_indices[1].reshape(CROWS, CHUNK)
    val = adj_values.reshape(CROWS, CHUNK)
    emb = jnp.concatenate(
        [user_emb, item_emb, jnp.zeros((N_PAD - N, D), jnp.float32)], axis=0)

    total = emb
    cur = emb
    for _ in range(N_LAYERS):
        partials = _spmm_layer(row, col, val, cur)
        cur = partials[0] + partials[1]
        total = total + cur

    final = total * (1.0 / (N_LAYERS + 1))
    return final[:N_USERS], final[N_USERS:N]


# speedup vs baseline: 2.2087x; 1.0434x over previous
import dataclasses
import functools

import jax
import jax.numpy as jnp
from jax import lax
from jax.experimental import pallas as pl
from jax.experimental.pallas import tpu as pltpu
from jax.experimental.pallas import tpu_sc as plsc

N_USERS = 25000
N_ITEMS = 25000
N = N_USERS + N_ITEMS
D = 32
N_LAYERS = 3
NNZ = 1600000

NC = 2
NS = 16
L = 16
NW = NC * NS

CHUNK = 80
CROWS = NNZ // CHUNK
CROWS_PER_W = CROWS // NW
K = 25
NUM_BLOCKS = CROWS_PER_W // K
NBUF = 5
N_PAD = 50048
ROWS_PER_SUB = N_PAD // NS

_MESH = plsc.VectorSubcoreMesh(core_axis_name="c", subcore_axis_name="s")

_CP = pltpu.CompilerParams(use_tc_tiling_on_sc=False)
if "needs_layout_passes" in pltpu.CompilerParams.__dataclass_fields__:
    _CP = dataclasses.replace(_CP, needs_layout_passes=False)


def _layer_body(row_hbm, col_hbm, val_hbm, emb_hbm, out_hbm,
                cb0, cb1, rb0, rb1, vb0, vb1, g0, g1, g2, g3, g4, acc,
                gs0, gs1, gs2, gs3, gs4, ss0, ss1, ss2, ss3, ss4,
                is0, is1, zsem):
    cid = lax.axis_index("c")
    sid = lax.axis_index("s")
    wid = cid * NS + sid

    colb = (cb0, cb1)
    rowb = (rb0, rb1)
    valb = (vb0, vb1)
    gbuf = (g0, g1, g2, g3, g4)
    gsem = (gs0, gs1, gs2, gs3, gs4)
    ssem = (ss0, ss1, ss2, ss3, ss4)
    isem = (is0, is1)

    cb00 = wid * CROWS_PER_W
    pltpu.async_copy(row_hbm.at[pl.ds(cb00, K)], rowb[0], isem[0])
    pltpu.async_copy(col_hbm.at[pl.ds(cb00, K)], colb[0], isem[0])
    pltpu.async_copy(val_hbm.at[pl.ds(cb00, K)], valb[0], isem[0])

    zero = jnp.zeros((L,), jnp.float32)
    for b in range(NBUF):
        g = gbuf[b]

        @pl.loop(0, CHUNK)
        def _(i):
            g[i, pl.ds(0, L)] = zero
            g[i, pl.ds(L, L)] = zero

    abase = sid * ROWS_PER_SUB
    nz = ROWS_PER_SUB // CHUNK
    for k in range(nz):
        pltpu.async_copy(gbuf[k % NBUF],
                         acc.at[pl.ds(abase + k * CHUNK, CHUNK)], zsem)
    rem = ROWS_PER_SUB - nz * CHUNK
    pltpu.async_copy(gbuf[0].at[pl.ds(0, rem)],
                     acc.at[pl.ds(abase + nz * CHUNK, rem)], zsem)
    for k in range(nz):
        pltpu.make_async_copy(gbuf[k % NBUF],
                              acc.at[pl.ds(abase, CHUNK)], zsem).wait()
    pltpu.make_async_copy(gbuf[0].at[pl.ds(0, rem)],
                          acc.at[pl.ds(abase, rem)], zsem).wait()

    plsc.subcore_barrier()

    def gissue(p, c, b):
        pltpu.async_copy(emb_hbm.at[colb[p].at[c]], gbuf[b], gsem[b])

    def gwait(b):
        pltpu.make_async_copy(emb_hbm.at[colb[0].at[0]], gbuf[b],
                              gsem[b]).wait()

    def sissue(p, c, b):
        pltpu.async_copy(gbuf[b], acc.at[rowb[p].at[c]], ssem[b], add=True)

    def swait(b):
        pltpu.make_async_copy(gbuf[b], acc.at[rowb[0].at[0]], ssem[b]).wait()

    def iissue(p, blk):
        cb = wid * CROWS_PER_W + blk * K
        pltpu.async_copy(row_hbm.at[pl.ds(cb, K)], rowb[p], isem[p])
        pltpu.async_copy(col_hbm.at[pl.ds(cb, K)], colb[p], isem[p])
        pltpu.async_copy(val_hbm.at[pl.ds(cb, K)], valb[p], isem[p])

    def iwait(p):
        pltpu.make_async_copy(row_hbm.at[pl.ds(0, K)], rowb[p], isem[p]).wait()
        pltpu.make_async_copy(col_hbm.at[pl.ds(0, K)], colb[p], isem[p]).wait()
        pltpu.make_async_copy(val_hbm.at[pl.ds(0, K)], valb[p], isem[p]).wait()

    def mul(p, c, b):
        g = gbuf[b]
        vb = valb[p]

        @plsc.parallel_loop(0, CHUNK, step=L, unroll=2)
        def _(e0):
            vv = vb[c, pl.ds(e0, L)]
            for i in range(L):
                v = vv.at[jnp.full((L,), i, jnp.int32)].get(
                    mode="promise_in_bounds")
                e = e0 + i
                g[e, pl.ds(0, L)] = g[e, pl.ds(0, L)] * v
                g[e, pl.ds(L, L)] = g[e, pl.ds(L, L)] * v

    def body(blk, p, last):
        if not last:
            iissue(1 - p, blk + 1)

        @pl.loop(0, K - NBUF, step=NBUF)
        def _(c0):
            for b in range(NBUF):
                gwait(b)
                mul(p, c0 + b, b)
                sissue(p, c0 + b, b)
            for b in range(NBUF):
                swait(b)
                gissue(p, c0 + NBUF + b, b)

        for b in range(NBUF):
            gwait(b)
            mul(p, K - NBUF + b, b)
            sissue(p, K - NBUF + b, b)
        if not last:
            iwait(1 - p)
            for b in range(NBUF):
                swait(b)
                gissue(1 - p, b, b)
        else:
            for b in range(NBUF):
                swait(b)

    iwait(0)
    for b in range(NBUF):
        gissue(0, b, b)

    @pl.loop(0, NUM_BLOCKS - 1, step=2)
    def _(blk):
        body(blk, 0, False)
        body(blk + 1, 1, False)

    body(NUM_BLOCKS - 1, 0, True)

    plsc.subcore_barrier()

    pltpu.sync_copy(acc.at[pl.ds(sid * ROWS_PER_SUB, ROWS_PER_SUB)],
                    out_hbm.at[cid].at[pl.ds(sid * ROWS_PER_SUB, ROWS_PER_SUB)])


@functools.partial(
    pl.kernel,
    out_type=jax.ShapeDtypeStruct((NC, N_PAD, D), jnp.float32),
    mesh=_MESH,
    scratch_types=(
        [pltpu.VMEM((K, CHUNK), jnp.int32)] * 2
        + [pltpu.VMEM((K, CHUNK), jnp.int32)] * 2
        + [pltpu.VMEM((K, CHUNK), jnp.float32)] * 2
        + [pltpu.VMEM((CHUNK, D), jnp.float32)] * NBUF
        + [pltpu.VMEM_SHARED((N_PAD, D), jnp.float32)]
        + [pltpu.SemaphoreType.DMA] * (2 * NBUF + 3)
    ),
    compiler_params=_CP,
)
def _spmm_layer(row_hbm, col_hbm, val_hbm, emb_hbm, out_hbm, *scratch):
    _layer_body(row_hbm, col_hbm, val_hbm, emb_hbm, out_hbm, *scratch)


def kernel(adj_indices, adj_values, user_emb, item_emb):
    row = adj_indices[0].reshape(CROWS, CHUNK)
    col = adj_indices[1].reshape(CROWS, CHUNK)
    val = adj_values.reshape(CROWS, CHUNK)
    emb = jnp.concatenate(
        [user_emb, item_emb, jnp.zeros((N_PAD - N, D), jnp.float32)], axis=0)

    total = emb
    cur = emb
    for _ in range(N_LAYERS):
        partials = _spmm_layer(row, col, val, cur)
        cur = partials[0] + partials[1]
        total = total + cur

    final = total * (1.0 / (N_LAYERS + 1))
    return final[:N_USERS], final[N_USERS:N]
